# baseline (speedup 1.0000x reference)
import functools

import jax
import jax.numpy as jnp
from jax import lax
from jax.experimental import pallas as pl
from jax.experimental.pallas import tpu as pltpu
from jax.experimental.pallas import tpu_sc as plsc

NC = 2
NS = 16
NW = NC * NS
LANES = 16
UNROLL = 8


def kernel(x, pos_emb):
    B, S, D = x.shape
    RW = S // NW
    R = min(32, RW)
    NP = RW // R
    CW = R * D
    NSTEP = NP * B

    xf = x.reshape(B * S * D)
    pf = pos_emb.reshape(-1)

    mesh = plsc.VectorSubcoreMesh(core_axis_name="c", subcore_axis_name="s")

    @functools.partial(
        pl.kernel,
        out_type=jax.ShapeDtypeStruct((B * S * D,), jnp.float32),
        mesh=mesh,
        scratch_types=(
            [pltpu.VMEM((CW,), jnp.float32) for _ in range(3)]
            + [pltpu.SemaphoreType.DMA for _ in range(5)]
        ),
    )
    def run(x_hbm, pos_hbm, out_hbm,
            xb0, xb1, pb,
            si0, si1, so0, so1, sp):
        xbufs = [xb0, xb1]
        sin = [si0, si1]
        sout = [so0, so1]

        c = lax.axis_index("c")
        s = lax.axis_index("s")
        wid = s * NC + c
        seq0 = wid * RW

        def xoff(step):
            p, b = divmod(step, B)
            return (b * S + seq0 + p * R) * D

        def start_xload(step):
            return pltpu.async_copy(
                x_hbm.at[pl.ds(xoff(step), CW)], xbufs[step % 2], sin[step % 2])

        def start_pload(p):
            return pltpu.async_copy(
                pos_hbm.at[pl.ds((seq0 + p * R) * D, CW)], pb, sp)

        ploads = {0: start_pload(0)}
        xloads = {0: start_xload(0), 1: start_xload(1)}
        stores = {}

        for step in range(NSTEP):
            p, b = divmod(step, B)
            xb = xbufs[step % 2]

            if b == 0:
                ploads.pop(p).wait()

            xloads[step].wait()

            def vbody(j, carry):
                for u in range(UNROLL):
                    sl = pl.ds((j * UNROLL + u) * LANES, LANES)
                    plsc.addupdate(xb.at[sl], pb[sl])
                return carry

            lax.fori_loop(0, CW // (LANES * UNROLL), vbody, 0)

            stores[step] = pltpu.async_copy(
                xb, out_hbm.at[pl.ds(xoff(step), CW)], sout[step % 2])

            if b == B - 1 and p + 1 < NP:
                ploads[p + 1] = start_pload(p + 1)

            if step + 2 < NSTEP:
                stores.pop(step).wait()
                xloads[step + 2] = start_xload(step + 2)

        for st in stores.values():
            st.wait()

    out = run(xf, pf)
    return out.reshape(B, S, D)


# speedup vs baseline: 4.4341x; 4.4341x over previous
import jax
import jax.numpy as jnp
from jax.experimental import pallas as pl

BS = 1024


def _add_kernel(x_ref, p_ref, o_ref):
    o_ref[...] = x_ref[...] + p_ref[...]


def kernel(x, pos_emb):
    B, S, D = x.shape
    grid = (S // BS, B)
    return pl.pallas_call(
        _add_kernel,
        grid=grid,
        in_specs=[
            pl.BlockSpec((1, BS, D), lambda i, b: (b, i, 0)),
            pl.BlockSpec((BS, D), lambda i, b: (i, 0)),
        ],
        out_specs=pl.BlockSpec((1, BS, D), lambda i, b: (b, i, 0)),
        out_shape=jax.ShapeDtypeStruct((B, S, D), jnp.float32),
    )(x, pos_emb)
